# baseline (speedup 1.0000x reference)
import functools

import jax
import jax.numpy as jnp
from jax import lax
from jax.experimental import pallas as pl
from jax.experimental.pallas import tpu as pltpu
from jax.experimental.pallas import tpu_sc as plsc

NN = 10000
EE = 320000
DH = 16

NC = 2
NS = 16
CH = 80
EPT = EE // (NC * NS)
NCHUNK = EPT // CH
RPT = NN // NS

_mesh = plsc.VectorSubcoreMesh(core_axis_name="c", subcore_axis_name="s")


def _zero_fill(buf, nrows):
    def body(i, _):
        buf[i, :] = jnp.zeros((DH,), jnp.float32)
        return _
    lax.fori_loop(0, nrows, body, None)


def _deg_body(ei_hbm, out_hbm, idx_d, ones_v, stage, zsh):
    cid = lax.axis_index("c")
    sid = lax.axis_index("s")
    wid = cid * NS + sid

    def fill_ones(i, _):
        ones_v[i, :] = jnp.full((DH,), 1.0, jnp.float32)
        return _
    lax.fori_loop(0, CH, fill_ones, None)

    _zero_fill(stage, RPT)
    pltpu.sync_copy(stage, zsh.at[pl.ds(sid * RPT, RPT)])
    pltpu.sync_copy(ei_hbm.at[1, wid], idx_d)
    plsc.subcore_barrier()

    def chunk(i, _):
        pltpu.sync_copy(ones_v, zsh.at[idx_d.at[i]], add=True)
        return _
    lax.fori_loop(0, NCHUNK, chunk, None)
    plsc.subcore_barrier()

    pltpu.sync_copy(zsh.at[pl.ds(sid * RPT, RPT)], stage)
    pltpu.sync_copy(stage, out_hbm.at[cid, sid])


_deg_call = pl.kernel(
    _deg_body,
    out_type=jax.ShapeDtypeStruct((NC, NS, RPT, DH), jnp.float32),
    mesh=_mesh,
    compiler_params=pltpu.CompilerParams(use_tc_tiling_on_sc=False),
    scratch_types=[
        pltpu.VMEM((NCHUNK, CH), jnp.int32),
        pltpu.VMEM((CH, DH), jnp.float32),
        pltpu.VMEM((RPT, DH), jnp.float32),
        pltpu.VMEM_SHARED((NN, DH), jnp.float32),
    ],
)


def _scat_body(y_hbm, ei_hbm, out_hbm, idx_s, idx_d, rows0, rows1,
               stage, zsh, ybuf, sem0, sem1):
    cid = lax.axis_index("c")
    sid = lax.axis_index("s")
    wid = cid * NS + sid

    _zero_fill(stage, RPT)
    pltpu.sync_copy(stage, zsh.at[pl.ds(sid * RPT, RPT)])
    pltpu.sync_copy(y_hbm.at[pl.ds(sid * RPT, RPT)],
                    ybuf.at[pl.ds(sid * RPT, RPT)])
    pltpu.sync_copy(ei_hbm.at[0, wid], idx_s)
    pltpu.sync_copy(ei_hbm.at[1, wid], idx_d)
    plsc.subcore_barrier()

    pltpu.async_copy(ybuf.at[idx_s.at[0]], rows0, sem0)

    def pair(j, _):
        i0 = 2 * j
        pltpu.make_async_copy(ybuf.at[idx_s.at[i0]], rows0, sem0).wait()
        pltpu.async_copy(ybuf.at[idx_s.at[i0 + 1]], rows1, sem1)
        pltpu.sync_copy(rows0, zsh.at[idx_d.at[i0]], add=True)
        pltpu.make_async_copy(ybuf.at[idx_s.at[i0 + 1]], rows1, sem1).wait()
        pltpu.async_copy(ybuf.at[idx_s.at[i0 + 2]], rows0, sem0)
        pltpu.sync_copy(rows1, zsh.at[idx_d.at[i0 + 1]], add=True)
        return _
    lax.fori_loop(0, (NCHUNK - 1) // 2, pair, None)

    pltpu.make_async_copy(ybuf.at[idx_s.at[NCHUNK - 1]], rows0, sem0).wait()
    pltpu.sync_copy(rows0, zsh.at[idx_d.at[NCHUNK - 1]], add=True)
    plsc.subcore_barrier()

    pltpu.sync_copy(zsh.at[pl.ds(sid * RPT, RPT)], stage)
    pltpu.sync_copy(stage, out_hbm.at[cid, sid])


_scat_call = pl.kernel(
    _scat_body,
    out_type=jax.ShapeDtypeStruct((NC, NS, RPT, DH), jnp.float32),
    mesh=_mesh,
    compiler_params=pltpu.CompilerParams(use_tc_tiling_on_sc=False),
    scratch_types=[
        pltpu.VMEM((NCHUNK, CH), jnp.int32),
        pltpu.VMEM((NCHUNK, CH), jnp.int32),
        pltpu.VMEM((CH, DH), jnp.float32),
        pltpu.VMEM((CH, DH), jnp.float32),
        pltpu.VMEM((RPT, DH), jnp.float32),
        pltpu.VMEM_SHARED((NN, DH), jnp.float32),
        pltpu.VMEM_SHARED((NN, DH), jnp.float32),
        pltpu.SemaphoreType.DMA,
        pltpu.SemaphoreType.DMA,
    ],
)


def _scat2_body(y1_hbm, z1p_hbm, dinv_hbm, b1_hbm, ei_hbm,
                out_hbm, y2_hbm, idx_s, idx_d, rows0, rows1, stage,
                sa, sb, sc_, sd, b1v, zsh, ybuf, sem0, sem1):
    cid = lax.axis_index("c")
    sid = lax.axis_index("s")
    wid = cid * NS + sid

    _zero_fill(stage, RPT)
    pltpu.sync_copy(stage, zsh.at[pl.ds(sid * RPT, RPT)])
    sl = pl.ds(sid * RPT, RPT)
    pltpu.sync_copy(y1_hbm.at[sl], sa)
    pltpu.sync_copy(z1p_hbm.at[0, sid], sb)
    pltpu.sync_copy(z1p_hbm.at[1, sid], sc_)
    pltpu.sync_copy(dinv_hbm.at[sl], sd)
    pltpu.sync_copy(b1_hbm, b1v)
    pltpu.sync_copy(ei_hbm.at[0, wid], idx_s)
    pltpu.sync_copy(ei_hbm.at[1, wid], idx_d)

    bias = b1v[...]

    def node(i, _):
        z = sa[i, :] + sb[i, :] + sc_[i, :]
        dv = sd[i, :]
        h = jnp.maximum(dv * z + bias, 0.0)
        sa[i, :] = dv * h
        return _
    lax.fori_loop(0, RPT, node, None)
    pltpu.sync_copy(sa, ybuf.at[sl])

    @pl.when(cid == 0)
    def _():
        pltpu.sync_copy(sa, y2_hbm.at[sid])
    plsc.subcore_barrier()

    pltpu.async_copy(ybuf.at[idx_s.at[0]], rows0, sem0)

    def pair(j, _):
        i0 = 2 * j
        pltpu.make_async_copy(ybuf.at[idx_s.at[i0]], rows0, sem0).wait()
        pltpu.async_copy(ybuf.at[idx_s.at[i0 + 1]], rows1, sem1)
        pltpu.sync_copy(rows0, zsh.at[idx_d.at[i0]], add=True)
        pltpu.make_async_copy(ybuf.at[idx_s.at[i0 + 1]], rows1, sem1).wait()
        pltpu.async_copy(ybuf.at[idx_s.at[i0 + 2]], rows0, sem0)
        pltpu.sync_copy(rows1, zsh.at[idx_d.at[i0 + 1]], add=True)
        return _
    lax.fori_loop(0, (NCHUNK - 1) // 2, pair, None)

    pltpu.make_async_copy(ybuf.at[idx_s.at[NCHUNK - 1]], rows0, sem0).wait()
    pltpu.sync_copy(rows0, zsh.at[idx_d.at[NCHUNK - 1]], add=True)
    plsc.subcore_barrier()

    pltpu.sync_copy(zsh.at[pl.ds(sid * RPT, RPT)], stage)
    pltpu.sync_copy(stage, out_hbm.at[cid, sid])


_scat2_call = pl.kernel(
    _scat2_body,
    out_type=[
        jax.ShapeDtypeStruct((NC, NS, RPT, DH), jnp.float32),
        jax.ShapeDtypeStruct((NS, RPT, DH), jnp.float32),
    ],
    mesh=_mesh,
    compiler_params=pltpu.CompilerParams(use_tc_tiling_on_sc=False),
    scratch_types=[
        pltpu.VMEM((NCHUNK, CH), jnp.int32),
        pltpu.VMEM((NCHUNK, CH), jnp.int32),
        pltpu.VMEM((CH, DH), jnp.float32),
        pltpu.VMEM((CH, DH), jnp.float32),
        pltpu.VMEM((RPT, DH), jnp.float32),
        pltpu.VMEM((RPT, DH), jnp.float32),
        pltpu.VMEM((RPT, DH), jnp.float32),
        pltpu.VMEM((RPT, DH), jnp.float32),
        pltpu.VMEM((RPT, DH), jnp.float32),
        pltpu.VMEM((DH,), jnp.float32),
        pltpu.VMEM_SHARED((NN, DH), jnp.float32),
        pltpu.VMEM_SHARED((NN, DH), jnp.float32),
        pltpu.SemaphoreType.DMA,
        pltpu.SemaphoreType.DMA,
    ],
)


def _prep_body(x_ref, w1_ref, degp_ref, y1_ref, dinv_ref):
    xw = jnp.dot(x_ref[...], w1_ref[...], preferred_element_type=jnp.float32)
    deg = degp_ref[0] + degp_ref[1] + 1.0
    dinv = lax.rsqrt(deg)
    dinv_ref[...] = dinv
    y1_ref[...] = dinv * xw


_PB = 1000
_prep_call = pl.pallas_call(
    _prep_body,
    grid=(NN // _PB,),
    in_specs=[
        pl.BlockSpec((_PB, 128), lambda i: (i, 0)),
        pl.BlockSpec((128, DH), lambda i: (0, 0)),
        pl.BlockSpec((NC, _PB, DH), lambda i: (0, i, 0)),
    ],
    out_specs=[
        pl.BlockSpec((_PB, DH), lambda i: (i, 0)),
        pl.BlockSpec((_PB, DH), lambda i: (i, 0)),
    ],
    out_shape=[
        jax.ShapeDtypeStruct((NN, DH), jnp.float32),
        jax.ShapeDtypeStruct((NN, DH), jnp.float32),
    ],
)


def _fin_body(y2_ref, z_ref, dinv_ref, w2_ref, b2_ref, o_ref):
    t = dinv_ref[...] * (y2_ref[...] + z_ref[0] + z_ref[1])
    h = jnp.dot(t, w2_ref[...], preferred_element_type=jnp.float32) + b2_ref[...]
    m = jnp.max(h, axis=1, keepdims=True)
    s = h - m
    lse = jnp.log(jnp.sum(jnp.exp(s), axis=1, keepdims=True))
    o_ref[...] = s - lse


_fin_call = pl.pallas_call(
    _fin_body,
    grid=(NN // _PB,),
    in_specs=[
        pl.BlockSpec((_PB, DH), lambda i: (i, 0)),
        pl.BlockSpec((NC, _PB, DH), lambda i: (0, i, 0)),
        pl.BlockSpec((_PB, DH), lambda i: (i, 0)),
        pl.BlockSpec((DH, 2), lambda i: (0, 0)),
        pl.BlockSpec((1, 2), lambda i: (0, 0)),
    ],
    out_specs=pl.BlockSpec((_PB, 2), lambda i: (i, 0)),
    out_shape=jax.ShapeDtypeStruct((NN, 2), jnp.float32),
)


def kernel(x, edge_index, W1, b1, W2, b2):
    ei = edge_index.reshape(2, NC * NS, NCHUNK, CH)
    degp = _deg_call(ei).reshape(NC, NN, DH)
    y1, dinv = _prep_call(x, W1, degp)
    z1 = _scat_call(y1, ei)
    z2, y2 = _scat2_call(y1, z1, dinv, b1, ei)
    return _fin_call(y2.reshape(NN, DH), z2.reshape(NC, NN, DH), dinv, W2,
                     b2.reshape(1, 2))


# speedup vs baseline: 1.0715x; 1.0715x over previous
import functools

import jax
import jax.numpy as jnp
from jax import lax
from jax.experimental import pallas as pl
from jax.experimental.pallas import tpu as pltpu
from jax.experimental.pallas import tpu_sc as plsc

NN = 10000
EE = 320000
DH = 16

NC = 2
NS = 16
CH = 80
EPT = EE // (NC * NS)
NCHUNK = EPT // CH
RPT = NN // NS

_mesh = plsc.VectorSubcoreMesh(core_axis_name="c", subcore_axis_name="s")


def _zero_fill(buf, nrows):
    def body(i, _):
        buf[i, :] = jnp.zeros((DH,), jnp.float32)
        return _
    lax.fori_loop(0, nrows, body, None)


def _deg_body(ei_hbm, out_hbm, idx_d, ones_v, stage, zsh, sem):
    cid = lax.axis_index("c")
    sid = lax.axis_index("s")
    wid = cid * NS + sid

    def fill_ones(i, _):
        ones_v[i, :] = jnp.full((DH,), 1.0, jnp.float32)
        return _
    lax.fori_loop(0, CH, fill_ones, None)

    _zero_fill(stage, RPT)
    pltpu.sync_copy(stage, zsh.at[pl.ds(sid * RPT, RPT)])
    pltpu.sync_copy(ei_hbm.at[1, wid], idx_d)
    plsc.subcore_barrier()

    for k in range(4):
        pltpu.async_copy(ones_v, zsh.at[idx_d.at[k]], sem, add=True)

    def chunk(i, _):
        pltpu.make_async_copy(ones_v, zsh.at[idx_d.at[i]], sem).wait()
        pltpu.async_copy(ones_v, zsh.at[idx_d.at[i + 4]], sem, add=True)
        return _
    lax.fori_loop(0, NCHUNK - 4, chunk, None)
    for k in range(4):
        pltpu.make_async_copy(ones_v, zsh.at[idx_d.at[0]], sem).wait()
    plsc.subcore_barrier()

    pltpu.sync_copy(zsh.at[pl.ds(sid * RPT, RPT)], stage)
    pltpu.sync_copy(stage, out_hbm.at[cid, sid])


_deg_call = pl.kernel(
    _deg_body,
    out_type=jax.ShapeDtypeStruct((NC, NS, RPT, DH), jnp.float32),
    mesh=_mesh,
    compiler_params=pltpu.CompilerParams(use_tc_tiling_on_sc=False),
    scratch_types=[
        pltpu.VMEM((NCHUNK, CH), jnp.int32),
        pltpu.VMEM((CH, DH), jnp.float32),
        pltpu.VMEM((RPT, DH), jnp.float32),
        pltpu.VMEM_SHARED((NN, DH), jnp.float32),
        pltpu.SemaphoreType.DMA,
    ],
)


def _scat_body(y_hbm, ei_hbm, out_hbm, idx_s, idx_d, rows0, rows1,
               stage, zsh, ybuf, sem0, sem1):
    cid = lax.axis_index("c")
    sid = lax.axis_index("s")
    wid = cid * NS + sid

    _zero_fill(stage, RPT)
    pltpu.sync_copy(stage, zsh.at[pl.ds(sid * RPT, RPT)])
    pltpu.sync_copy(y_hbm.at[pl.ds(sid * RPT, RPT)],
                    ybuf.at[pl.ds(sid * RPT, RPT)])
    pltpu.sync_copy(ei_hbm.at[0, wid], idx_s)
    pltpu.sync_copy(ei_hbm.at[1, wid], idx_d)
    plsc.subcore_barrier()

    pltpu.async_copy(ybuf.at[idx_s.at[0]], rows0, sem0)

    def pair(j, _):
        i0 = 2 * j
        pltpu.make_async_copy(ybuf.at[idx_s.at[i0]], rows0, sem0).wait()
        pltpu.async_copy(ybuf.at[idx_s.at[i0 + 1]], rows1, sem1)
        pltpu.sync_copy(rows0, zsh.at[idx_d.at[i0]], add=True)
        pltpu.make_async_copy(ybuf.at[idx_s.at[i0 + 1]], rows1, sem1).wait()
        pltpu.async_copy(ybuf.at[idx_s.at[i0 + 2]], rows0, sem0)
        pltpu.sync_copy(rows1, zsh.at[idx_d.at[i0 + 1]], add=True)
        return _
    lax.fori_loop(0, (NCHUNK - 1) // 2, pair, None)

    pltpu.make_async_copy(ybuf.at[idx_s.at[NCHUNK - 1]], rows0, sem0).wait()
    pltpu.sync_copy(rows0, zsh.at[idx_d.at[NCHUNK - 1]], add=True)
    plsc.subcore_barrier()

    pltpu.sync_copy(zsh.at[pl.ds(sid * RPT, RPT)], stage)
    pltpu.sync_copy(stage, out_hbm.at[cid, sid])


_scat_call = pl.kernel(
    _scat_body,
    out_type=jax.ShapeDtypeStruct((NC, NS, RPT, DH), jnp.float32),
    mesh=_mesh,
    compiler_params=pltpu.CompilerParams(use_tc_tiling_on_sc=False),
    scratch_types=[
        pltpu.VMEM((NCHUNK, CH), jnp.int32),
        pltpu.VMEM((NCHUNK, CH), jnp.int32),
        pltpu.VMEM((CH, DH), jnp.float32),
        pltpu.VMEM((CH, DH), jnp.float32),
        pltpu.VMEM((RPT, DH), jnp.float32),
        pltpu.VMEM_SHARED((NN, DH), jnp.float32),
        pltpu.VMEM_SHARED((NN, DH), jnp.float32),
        pltpu.SemaphoreType.DMA,
        pltpu.SemaphoreType.DMA,
    ],
)


def _scat2_body(y1_hbm, z1p_hbm, dinv_hbm, b1_hbm, ei_hbm,
                out_hbm, y2_hbm, idx_s, idx_d, rows0, rows1, stage,
                sa, sb, sc_, sd, b1v, zsh, ybuf, sem0, sem1):
    cid = lax.axis_index("c")
    sid = lax.axis_index("s")
    wid = cid * NS + sid

    _zero_fill(stage, RPT)
    pltpu.sync_copy(stage, zsh.at[pl.ds(sid * RPT, RPT)])
    sl = pl.ds(sid * RPT, RPT)
    pltpu.sync_copy(y1_hbm.at[sl], sa)
    pltpu.sync_copy(z1p_hbm.at[0, sid], sb)
    pltpu.sync_copy(z1p_hbm.at[1, sid], sc_)
    pltpu.sync_copy(dinv_hbm.at[sl], sd)
    pltpu.sync_copy(b1_hbm, b1v)
    pltpu.sync_copy(ei_hbm.at[0, wid], idx_s)
    pltpu.sync_copy(ei_hbm.at[1, wid], idx_d)

    bias = b1v[...]

    def node(i, _):
        z = sa[i, :] + sb[i, :] + sc_[i, :]
        dv = sd[i, :]
        h = jnp.maximum(dv * z + bias, 0.0)
        sa[i, :] = dv * h
        return _
    lax.fori_loop(0, RPT, node, None)
    pltpu.sync_copy(sa, ybuf.at[sl])

    @pl.when(cid == 0)
    def _():
        pltpu.sync_copy(sa, y2_hbm.at[sid])
    plsc.subcore_barrier()

    pltpu.async_copy(ybuf.at[idx_s.at[0]], rows0, sem0)

    def pair(j, _):
        i0 = 2 * j
        pltpu.make_async_copy(ybuf.at[idx_s.at[i0]], rows0, sem0).wait()
        pltpu.async_copy(ybuf.at[idx_s.at[i0 + 1]], rows1, sem1)
        pltpu.sync_copy(rows0, zsh.at[idx_d.at[i0]], add=True)
        pltpu.make_async_copy(ybuf.at[idx_s.at[i0 + 1]], rows1, sem1).wait()
        pltpu.async_copy(ybuf.at[idx_s.at[i0 + 2]], rows0, sem0)
        pltpu.sync_copy(rows1, zsh.at[idx_d.at[i0 + 1]], add=True)
        return _
    lax.fori_loop(0, (NCHUNK - 1) // 2, pair, None)

    pltpu.make_async_copy(ybuf.at[idx_s.at[NCHUNK - 1]], rows0, sem0).wait()
    pltpu.sync_copy(rows0, zsh.at[idx_d.at[NCHUNK - 1]], add=True)
    plsc.subcore_barrier()

    pltpu.sync_copy(zsh.at[pl.ds(sid * RPT, RPT)], stage)
    pltpu.sync_copy(stage, out_hbm.at[cid, sid])


_scat2_call = pl.kernel(
    _scat2_body,
    out_type=[
        jax.ShapeDtypeStruct((NC, NS, RPT, DH), jnp.float32),
        jax.ShapeDtypeStruct((NS, RPT, DH), jnp.float32),
    ],
    mesh=_mesh,
    compiler_params=pltpu.CompilerParams(use_tc_tiling_on_sc=False),
    scratch_types=[
        pltpu.VMEM((NCHUNK, CH), jnp.int32),
        pltpu.VMEM((NCHUNK, CH), jnp.int32),
        pltpu.VMEM((CH, DH), jnp.float32),
        pltpu.VMEM((CH, DH), jnp.float32),
        pltpu.VMEM((RPT, DH), jnp.float32),
        pltpu.VMEM((RPT, DH), jnp.float32),
        pltpu.VMEM((RPT, DH), jnp.float32),
        pltpu.VMEM((RPT, DH), jnp.float32),
        pltpu.VMEM((RPT, DH), jnp.float32),
        pltpu.VMEM((DH,), jnp.float32),
        pltpu.VMEM_SHARED((NN, DH), jnp.float32),
        pltpu.VMEM_SHARED((NN, DH), jnp.float32),
        pltpu.SemaphoreType.DMA,
        pltpu.SemaphoreType.DMA,
    ],
)


def _prep_body(x_ref, w1_ref, degp_ref, y1_ref, dinv_ref):
    xw = jnp.dot(x_ref[...], w1_ref[...], preferred_element_type=jnp.float32)
    deg = degp_ref[0] + degp_ref[1] + 1.0
    dinv = lax.rsqrt(deg)
    dinv_ref[...] = dinv
    y1_ref[...] = dinv * xw


_prep_call = pl.pallas_call(
    _prep_body,
    out_shape=[
        jax.ShapeDtypeStruct((NN, DH), jnp.float32),
        jax.ShapeDtypeStruct((NN, DH), jnp.float32),
    ],
)


def _fin_body(y2_ref, z_ref, dinv_ref, w2_ref, b2_ref, o_ref):
    t = dinv_ref[...] * (y2_ref[...] + z_ref[0] + z_ref[1])
    h = jnp.dot(t, w2_ref[...], preferred_element_type=jnp.float32) + b2_ref[...]
    m = jnp.max(h, axis=1, keepdims=True)
    s = h - m
    lse = jnp.log(jnp.sum(jnp.exp(s), axis=1, keepdims=True))
    o_ref[...] = s - lse


_fin_call = pl.pallas_call(
    _fin_body,
    out_shape=jax.ShapeDtypeStruct((NN, 2), jnp.float32),
)


def kernel(x, edge_index, W1, b1, W2, b2):
    ei = edge_index.reshape(2, NC * NS, NCHUNK, CH)
    degp = _deg_call(ei).reshape(NC, NN, DH)
    y1, dinv = _prep_call(x, W1, degp)
    z1 = _scat_call(y1, ei)
    z2, y2 = _scat2_call(y1, z1, dinv, b1, ei)
    return _fin_call(y2.reshape(NN, DH), z2.reshape(NC, NN, DH), dinv, W2,
                     b2.reshape(1, 2))
